# baseline (speedup 1.0000x reference)
import functools
import math

import jax
import jax.numpy as jnp
from jax import lax
from jax.experimental import pallas as pl
from jax.experimental.pallas import tpu as pltpu
from jax.experimental.pallas import tpu_sc as plsc

BATCH = 4
SEQ_LEN = 8192
EMBED_DIM = 64
MAX_WAVELENGTH = 10000.0

_NC = 2
_NS = 16
_NW = _NC * _NS
_ROWS = BATCH * SEQ_LEN
_BW = _ROWS // _NW
_CH = 256
_NCHUNK = _BW // _CH
_LANES = EMBED_DIM // 16


def _enc_body(out_ref):
    pos = lax.broadcasted_iota(jnp.int32, (SEQ_LEN, EMBED_DIM), 0).astype(jnp.float32)
    col = lax.broadcasted_iota(jnp.int32, (SEQ_LEN, EMBED_DIM), 1)
    expo = (2 * (col // 2)).astype(jnp.float32) / float(EMBED_DIM)
    ln_base = -math.log(MAX_WAVELENGTH)
    timescales = jnp.exp(expo * ln_base)
    angles = pos * timescales
    odd = (col % 2).astype(jnp.float32)
    out_ref[...] = jnp.sin(angles) * (1.0 - odd) + jnp.cos(angles) * odd


def _position_encoding_tc():
    return pl.pallas_call(
        _enc_body,
        out_shape=jax.ShapeDtypeStruct((SEQ_LEN, EMBED_DIM), jnp.float32),
    )()


_mesh = plsc.VectorSubcoreMesh(core_axis_name="c", subcore_axis_name="s")


@functools.partial(
    pl.kernel,
    out_type=jax.ShapeDtypeStruct((_ROWS, EMBED_DIM), jnp.float32),
    mesh=_mesh,
    scratch_types=[
        pltpu.VMEM((_BW,), jnp.int32),
        pltpu.VMEM((_BW,), jnp.int32),
        pltpu.VMEM((_CH, 2 * EMBED_DIM), jnp.float32),
        pltpu.VMEM((_CH, EMBED_DIM), jnp.float32),
        pltpu.VMEM((_CH, EMBED_DIM), jnp.float32),
        pltpu.SemaphoreType.DMA,
    ],
)
def _gather_add(tab2_hbm, idx_hbm, enc_hbm, out_hbm,
                idx_v, pidx_v, rows_v, enc_v, out_v, sem):
    wid = lax.axis_index("s") * _NC + lax.axis_index("c")
    base = wid * _BW
    enc_base = base % SEQ_LEN
    pltpu.sync_copy(idx_hbm.at[pl.ds(base, _BW)], idx_v)
    for j in range(_BW // 16):
        sl = pl.ds(j * 16, 16)
        pidx_v[sl] = lax.shift_right_logical(idx_v[sl], 1)
    for ci in range(_NCHUNK):
        pltpu.async_copy(
            tab2_hbm.at[pidx_v.at[pl.ds(ci * _CH, _CH)]], rows_v, sem
        ).wait()
        pltpu.sync_copy(enc_hbm.at[pl.ds(enc_base + ci * _CH, _CH)], enc_v)

        def _add_block(b, carry):
            rbase = b * 16
            par = idx_v[pl.ds(ci * _CH + rbase, 16)] & 1
            for rr in range(16):
                off = par[rr] * EMBED_DIM
                r = rbase + rr
                for g in range(_LANES):
                    out_v[r, pl.ds(g * 16, 16)] = (
                        rows_v[r, pl.ds(off + g * 16, 16)]
                        + enc_v[r, pl.ds(g * 16, 16)]
                    )
            return carry

        lax.fori_loop(0, _CH // 16, _add_block, 0)
        pltpu.sync_copy(out_v, out_hbm.at[pl.ds(base + ci * _CH, _CH)])


def kernel(inputs, table):
    idx = inputs.reshape(-1).astype(jnp.int32)
    tab2 = table.reshape(table.shape[0] // 2, 2 * EMBED_DIM)
    enc = _position_encoding_tc()
    out = _gather_add(tab2, idx, enc)
    return out.reshape(BATCH, SEQ_LEN, EMBED_DIM)


# speedup vs baseline: 1.2490x; 1.2490x over previous
import functools
import math

import jax
import jax.numpy as jnp
from jax import lax
from jax.experimental import pallas as pl
from jax.experimental.pallas import tpu as pltpu
from jax.experimental.pallas import tpu_sc as plsc

BATCH = 4
SEQ_LEN = 8192
EMBED_DIM = 64
VOCAB = 1000000
HALF_V = VOCAB // 2
MAX_WAVELENGTH = 10000.0

_NC = 2
_NS = 16
_NW = _NC * _NS
_ROWS = BATCH * SEQ_LEN
_BW = _ROWS // _NW
_CH = 256
_NCHUNK = _BW // _CH
_LANES = EMBED_DIM // 16

_TCB = 1024


def _relayout_body(in_ref, out_ref):
    x = in_ref[...]
    lo = x[:, :_TCB]
    hi = x[:, _TCB:]
    out_ref[...] = jnp.concatenate([lo.T, hi.T], axis=1)


_RGRID = (VOCAB + 2 * _TCB - 1) // (2 * _TCB)


def _relayout_tc(tabt):
    return pl.pallas_call(
        _relayout_body,
        grid=(_RGRID,),
        in_specs=[pl.BlockSpec((EMBED_DIM, 2 * _TCB), lambda i: (0, i))],
        out_specs=pl.BlockSpec((_TCB, 2 * EMBED_DIM), lambda i: (i, 0)),
        out_shape=jax.ShapeDtypeStruct((_RGRID * _TCB, 2 * EMBED_DIM), jnp.float32),
    )(tabt)


def _enc_body(out_ref):
    pos = lax.broadcasted_iota(jnp.int32, (SEQ_LEN, EMBED_DIM), 0).astype(jnp.float32)
    col = lax.broadcasted_iota(jnp.int32, (SEQ_LEN, EMBED_DIM), 1)
    expo = (2 * (col // 2)).astype(jnp.float32) / float(EMBED_DIM)
    ln_base = -math.log(MAX_WAVELENGTH)
    timescales = jnp.exp(expo * ln_base)
    angles = pos * timescales
    odd = (col % 2).astype(jnp.float32)
    out_ref[...] = jnp.sin(angles) * (1.0 - odd) + jnp.cos(angles) * odd


def _position_encoding_tc():
    return pl.pallas_call(
        _enc_body,
        out_shape=jax.ShapeDtypeStruct((SEQ_LEN, EMBED_DIM), jnp.float32),
    )()


_mesh = plsc.VectorSubcoreMesh(core_axis_name="c", subcore_axis_name="s")


@functools.partial(
    pl.kernel,
    out_type=jax.ShapeDtypeStruct((_ROWS, EMBED_DIM), jnp.float32),
    mesh=_mesh,
    scratch_types=[
        pltpu.VMEM((_BW,), jnp.int32),
        pltpu.VMEM((_BW,), jnp.int32),
        pltpu.VMEM((_CH, 2 * EMBED_DIM), jnp.float32),
        pltpu.VMEM((_CH, EMBED_DIM), jnp.float32),
        pltpu.VMEM((_CH, EMBED_DIM), jnp.float32),
        pltpu.SemaphoreType.DMA,
    ],
)
def _gather_add(tabr_hbm, idx_hbm, enc_hbm, out_hbm,
                idx_v, pidx_v, rows_v, enc_v, out_v, sem):
    wid = lax.axis_index("s") * _NC + lax.axis_index("c")
    base = wid * _BW
    enc_base = base % SEQ_LEN
    pltpu.sync_copy(idx_hbm.at[pl.ds(base, _BW)], idx_v)
    for j in range(_BW // 16):
        sl = pl.ds(j * 16, 16)
        v = idx_v[sl]
        pidx_v[sl] = lax.shift_left(lax.shift_right_logical(v, 11), 10) | (v & 1023)
    for ci in range(_NCHUNK):
        pltpu.async_copy(
            tabr_hbm.at[pidx_v.at[pl.ds(ci * _CH, _CH)]], rows_v, sem
        ).wait()
        pltpu.sync_copy(enc_hbm.at[pl.ds(enc_base + ci * _CH, _CH)], enc_v)

        def _add_block(b, carry):
            rbase = b * 16
            hi = (lax.shift_right_logical(
                idx_v[pl.ds(ci * _CH + rbase, 16)], 10) & 1) * EMBED_DIM
            for rr in range(16):
                off = hi[rr]
                r = rbase + rr
                for g in range(_LANES):
                    out_v[r, pl.ds(g * 16, 16)] = (
                        rows_v[r, pl.ds(off + g * 16, 16)]
                        + enc_v[r, pl.ds(g * 16, 16)]
                    )
            return carry

        lax.fori_loop(0, _CH // 16, _add_block, 0)
        pltpu.sync_copy(out_v, out_hbm.at[pl.ds(base + ci * _CH, _CH)])


def kernel(inputs, table):
    idx = inputs.reshape(-1).astype(jnp.int32)
    tabt = jnp.swapaxes(table, 0, 1)
    tabr = _relayout_tc(tabt)
    enc = _position_encoding_tc()
    out = _gather_add(tabr, idx, enc)
    return out.reshape(BATCH, SEQ_LEN, EMBED_DIM)
